# baseline (speedup 1.0000x reference)
import jax
import jax.numpy as jnp
from jax import lax
from jax.experimental import pallas as pl
from jax.experimental.pallas import tpu as pltpu
from jax.experimental.pallas import tpu_sc as plsc

B, C, H, W = 8, 96, 224, 224
SPLIT = 4
R = B * C * SPLIT
D = H * W // SPLIT
NC, NS = 2, 16
NW = NC * NS
M = R // NW
NBUF = 8
K = 3


def _body(x_hbm, idx_hbm, out_hbm, idx_v, bufs, gsems, ssems):
    wid = lax.axis_index("s") * NC + lax.axis_index("c")
    base = wid * M
    pltpu.sync_copy(idx_hbm.at[pl.ds(base, M)], idx_v)

    def gather(j):
        b = j % NBUF
        return pltpu.async_copy(x_hbm.at[idx_v.at[j]], bufs[b], gsems[b])

    def wait_gather(j):
        b = j % NBUF
        pltpu.make_async_copy(x_hbm.at[idx_v.at[j]], bufs[b],
                              gsems[b]).wait()

    def scatter(j):
        b = j % NBUF
        return pltpu.async_copy(bufs[b], out_hbm.at[pl.ds(base + j, 1)],
                                ssems[b])

    def wait_scatter(j):
        b = j % NBUF
        pltpu.make_async_copy(bufs[b], out_hbm.at[pl.ds(base + j, 1)],
                              ssems[b]).wait()

    for j in range(NBUF):
        gather(j)
    for j in range(M):
        wait_gather(j)
        scatter(j)
        if j >= K and j - K + NBUF < M:
            wait_scatter(j - K)
            gather(j - K + NBUF)
    for j in range(M - K, M):
        wait_scatter(j)


@jax.jit
def kernel(x, perm):
    x2 = x.reshape(R, D)
    rows = jnp.arange(B * C, dtype=jnp.int32)
    src_row = (rows // C) * C + perm.astype(jnp.int32)[rows % C]
    sub = jnp.arange(SPLIT, dtype=jnp.int32)
    src = (src_row[:, None] * SPLIT + sub[None, :]).reshape(R, 1)

    mesh = plsc.VectorSubcoreMesh(core_axis_name="c", subcore_axis_name="s")
    out2 = pl.kernel(
        _body,
        out_type=jax.ShapeDtypeStruct((R, D), jnp.float32),
        mesh=mesh,
        scratch_types=[
            pltpu.VMEM((M, 1), jnp.int32),
            [pltpu.VMEM((1, D), jnp.float32) for _ in range(NBUF)],
            [pltpu.SemaphoreType.DMA for _ in range(NBUF)],
            [pltpu.SemaphoreType.DMA for _ in range(NBUF)],
        ],
    )(x2, src)
    return out2.reshape(B, C, H, W)


# speedup vs baseline: 3.3822x; 3.3822x over previous
import jax
import jax.numpy as jnp
from jax import lax
from jax.experimental import pallas as pl
from jax.experimental.pallas import tpu as pltpu
from jax.experimental.pallas import tpu_sc as plsc

B, C, H, W = 8, 96, 224, 224
R = B * C
NC, NS = 2, 16
NW = NC * NS
M = R // NW


def _body(x_hbm, idx_hbm, out_hbm, idx_v, bufs, gsems, ssems):
    wid = lax.axis_index("s") * NC + lax.axis_index("c")
    base = wid * M
    pltpu.sync_copy(idx_hbm.at[pl.ds(base, M)], idx_v)

    lanes = lax.broadcasted_iota(jnp.int32, (16,), 0)
    v0 = idx_v[pl.ds(0, 16)]
    v1 = idx_v[pl.ds(8, 16)]

    def src_of(j):
        vec, lane = (v0, j) if j < 16 else (v1, j - 8)
        return lax.reduce_max(jnp.where(lanes == lane, vec, 0), (0,))

    def gather(j):
        b = j % 2
        return pltpu.async_copy(x_hbm.at[pl.ds(src_of(j), 1)], bufs[b],
                                gsems[b])

    def wait_gather(j):
        b = j % 2
        pltpu.make_async_copy(x_hbm.at[pl.ds(src_of(j), 1)], bufs[b],
                              gsems[b]).wait()

    def scatter(j):
        b = j % 2
        return pltpu.async_copy(bufs[b], out_hbm.at[pl.ds(base + j, 1)],
                                ssems[b])

    def wait_scatter(j):
        b = j % 2
        pltpu.make_async_copy(bufs[b], out_hbm.at[pl.ds(base + j, 1)],
                              ssems[b]).wait()

    gather(0)
    gather(1)
    for j in range(M):
        wait_gather(j)
        scatter(j)
        if j + 2 < M:
            wait_scatter(j)
            gather(j + 2)
    wait_scatter(M - 2)
    wait_scatter(M - 1)


@jax.jit
def kernel(x, perm):
    x3 = x.reshape(R, H, W)
    rows = jnp.arange(R, dtype=jnp.int32)
    src = (rows // C) * C + perm.astype(jnp.int32)[rows % C]

    mesh = plsc.VectorSubcoreMesh(core_axis_name="c", subcore_axis_name="s")
    out3 = pl.kernel(
        _body,
        out_type=jax.ShapeDtypeStruct((R, H, W), jnp.float32),
        mesh=mesh,
        compiler_params=pltpu.CompilerParams(use_tc_tiling_on_sc=True,
                                             needs_layout_passes=False),
        scratch_types=[
            pltpu.VMEM((M,), jnp.int32),
            [pltpu.VMEM((1, H, W), jnp.float32) for _ in range(2)],
            [pltpu.SemaphoreType.DMA for _ in range(2)],
            [pltpu.SemaphoreType.DMA for _ in range(2)],
        ],
    )(x3, src)
    return out3.reshape(B, C, H, W)
